# Initial kernel scaffold; baseline (speedup 1.0000x reference)
import jax
import jax.numpy as jnp
from jax.experimental import pallas as pl


def kernel(x, edge_index, W1, b1, W2, b2, lin_W, lin_b):
    raise NotImplementedError("write your pallas kernel here")


# speedup vs baseline: 1.0001x; 1.0001x over previous
import jax
import jax.numpy as jnp
from jax.experimental import pallas as pl

N_NODES = 10000


def _cheb_conv(x, edge_index, W, b):
    src = edge_index[0]
    dst = edge_index[1]
    deg = jax.ops.segment_sum(jnp.ones((src.shape[0],), dtype=x.dtype), src,
                              num_segments=N_NODES)
    dinv = jnp.where(deg > 0, 1.0 / jnp.sqrt(deg), 0.0)
    norm = -dinv[src] * dinv[dst]

    def lhat(h):
        return jax.ops.segment_sum(norm[:, None] * jnp.take(h, src, axis=0), dst,
                                   num_segments=N_NODES)

    Tx0 = x
    out = Tx0 @ W[0]
    Tx1 = lhat(x)
    out = out + Tx1 @ W[1]
    for k in range(2, W.shape[0]):
        Tx2 = 2.0 * lhat(Tx1) - Tx0
        out = out + Tx2 @ W[k]
        Tx0, Tx1 = Tx2, Tx2
    return out + b


def _cheb_conv_fixed(x, edge_index, W, b):
    src = edge_index[0]
    dst = edge_index[1]
    deg = jax.ops.segment_sum(jnp.ones((src.shape[0],), dtype=x.dtype), src,
                              num_segments=N_NODES)
    dinv = jnp.where(deg > 0, 1.0 / jnp.sqrt(deg), 0.0)
    norm = -dinv[src] * dinv[dst]

    def lhat(h):
        return jax.ops.segment_sum(norm[:, None] * jnp.take(h, src, axis=0), dst,
                                   num_segments=N_NODES)

    Tx0 = x
    out = Tx0 @ W[0]
    Tx1 = lhat(x)
    out = out + Tx1 @ W[1]
    for k in range(2, W.shape[0]):
        Tx2 = 2.0 * lhat(Tx1) - Tx0
        out = out + Tx2 @ W[k]
        Tx0, Tx1 = Tx1, Tx2
    return out + b


def _head_kernel(pooled_ref, w_ref, b_ref, out_ref):
    out_ref[...] = pooled_ref[...] @ w_ref[...] + b_ref[...]


def kernel(x, edge_index, W1, b1, W2, b2, lin_W, lin_b):
    h = jax.nn.relu(_cheb_conv_fixed(x, edge_index, W1, b1))
    h = jax.nn.relu(_cheb_conv_fixed(h, edge_index, W2, b2))
    pooled = jnp.sum(h, axis=0, keepdims=True)
    out = pl.pallas_call(
        _head_kernel,
        out_shape=jax.ShapeDtypeStruct((1, lin_W.shape[1]), jnp.float32),
    )(pooled, lin_W, lin_b[None, :])
    return (pooled, out)


# speedup vs baseline: 4.9122x; 4.9115x over previous
import functools

import jax
import jax.numpy as jnp
from jax import lax
from jax.experimental import pallas as pl
from jax.experimental.pallas import tpu as pltpu
from jax.experimental.pallas import tpu_sc as plsc

N_NODES = 10000
N_PAD = 10240
E = 320000
E_PAD = 323584
EPW = E_PAD // 32
CHUNK = 128
NCH = EPW // CHUNK
RPW = N_PAD // 16
NB = 512
NBLK = N_PAD // NB
HR = N_PAD // CHUNK
FC = 128


def _mesh():
    return plsc.VectorSubcoreMesh(core_axis_name="c", subcore_axis_name="s")


def _make_scatter(stacked):
    epw = (E_PAD // 16) if stacked else EPW
    nch = epw // CHUNK

    @functools.partial(
        pl.kernel,
        out_type=jax.ShapeDtypeStruct((2 * N_PAD, FC), jnp.float32),
        mesh=_mesh(),
        scratch_types=[
            pltpu.VMEM((CHUNK, FC), jnp.float32),
            pltpu.VMEM((CHUNK,), jnp.int32),
            pltpu.VMEM((CHUNK,), jnp.int32),
            pltpu.VMEM_SHARED((N_PAD, FC), jnp.float32),
            pltpu.SemaphoreType.DMA,
        ],
    )
    def s_kernel(g, src_hbm, dst_hbm, out, rows_v, src_v, dst_v, acc, sem):
        c = lax.axis_index("c")
        s = lax.axis_index("s")
        wid = s if stacked else s * 2 + c

        z16 = jnp.zeros((16,), jnp.float32)

        def zrow(i, carry):
            for j in range(FC // 16):
                rows_v[i, pl.ds(j * 16, 16)] = z16
            return carry

        lax.fori_loop(0, CHUNK, zrow, 0)
        for t in range(RPW // CHUNK):
            pltpu.sync_copy(rows_v, acc.at[pl.ds(s * RPW + t * CHUNK, CHUNK)])
        plsc.subcore_barrier()

        src_base = c * E_PAD if stacked else 0

        def body(j, carry):
            base = wid * epw + j * CHUNK
            pltpu.sync_copy(src_hbm.at[pl.ds(src_base + base, CHUNK)], src_v)
            pltpu.sync_copy(dst_hbm.at[pl.ds(base, CHUNK)], dst_v)
            pltpu.async_copy(g.at[src_v], rows_v, sem).wait()
            pltpu.sync_copy(rows_v, acc.at[dst_v], add=True)
            return carry

        lax.fori_loop(0, nch, body, 0)
        plsc.subcore_barrier()

        for t in range(RPW // CHUNK):
            r0 = s * RPW + t * CHUNK
            pltpu.sync_copy(acc.at[pl.ds(r0, CHUNK)], rows_v)
            pltpu.sync_copy(rows_v, out.at[pl.ds(c * N_PAD + r0, CHUNK)])

    return s_kernel


_scatter_es = _make_scatter(False)
_scatter_cs = _make_scatter(True)


@functools.partial(
    pl.kernel,
    out_type=jax.ShapeDtypeStruct((2 * N_PAD,), jnp.float32),
    mesh=_mesh(),
    scratch_types=[
        pltpu.VMEM((RPW,), jnp.float32),
        pltpu.VMEM((CHUNK,), jnp.float32),
        pltpu.VMEM((CHUNK,), jnp.int32),
        pltpu.VMEM_SHARED((N_PAD,), jnp.float32),
        pltpu.SemaphoreType.DMA,
    ],
)
def _deg_kernel(src_hbm, deg_out, stage, ones_v, src_v, acc, sem):
    c = lax.axis_index("c")
    s = lax.axis_index("s")
    wid = s * 2 + c

    z16 = jnp.zeros((16,), jnp.float32)
    one16 = jnp.ones((16,), jnp.float32)

    def zh(i, carry):
        stage[pl.ds(i * 16, 16)] = z16
        return carry

    lax.fori_loop(0, RPW // 16, zh, 0)
    for u in range(CHUNK // 16):
        ones_v[pl.ds(u * 16, 16)] = one16

    pltpu.sync_copy(stage, acc.at[pl.ds(s * RPW, RPW)])
    plsc.subcore_barrier()

    def body(j, carry):
        base = wid * EPW + j * CHUNK
        pltpu.sync_copy(src_hbm.at[pl.ds(base, CHUNK)], src_v)
        pltpu.sync_copy(ones_v, acc.at[src_v], add=True)
        return carry

    lax.fori_loop(0, NCH, body, 0)
    plsc.subcore_barrier()

    r0 = s * RPW
    pltpu.sync_copy(acc.at[pl.ds(r0, RPW)], stage)
    pltpu.sync_copy(stage, deg_out.at[pl.ds(c * N_PAD + r0, RPW)])


def _dinv_body(da_ref, db_ref, o_ref):
    d = da_ref[...] + db_ref[...]
    o_ref[...] = jnp.where(d > 0, lax.rsqrt(d), 0.0)


def _premul0_body(x_ref, dv_ref, g_ref):
    dv = dv_ref[...]
    g_ref[...] = -(dv * x_ref[...])


def _premul_sum_body(sa_ref, sb_ref, dv_ref, g_ref):
    dv = dv_ref[...]
    g_ref[...] = -(dv * dv) * (sa_ref[...] + sb_ref[...])


def _q_sum_body(sa_ref, sb_ref, x_ref, dv_ref, g_ref):
    dv = dv_ref[...]
    g_ref[...] = -dv * (2.0 * dv * (sa_ref[...] + sb_ref[...]) - x_ref[...])


def _premul_stk_body(s_ref, dv_ref, g_ref):
    dv = dv_ref[...]
    g_ref[...] = -(dv * dv) * s_ref[...]


def _q_stk_body(s_ref, t_ref, dv_ref, g_ref):
    dv = dv_ref[...]
    g_ref[...] = -dv * (2.0 * dv * s_ref[...] - t_ref[...])


def _m1_body(x_ref, s1a_ref, s1b_ref, s2a_ref, s2b_ref,
             s3a_ref, s3b_ref, dv_ref, w_ref, b_ref,
             ha_ref, hb_ref, ga_ref, gb_ref):
    dv = dv_ref[...]
    tx0 = x_ref[...]
    tx1 = dv * (s1a_ref[...] + s1b_ref[...])
    tx2 = 2.0 * dv * (s2a_ref[...] + s2b_ref[...]) - tx0
    tx3 = 2.0 * dv * (s3a_ref[...] + s3b_ref[...]) - tx1
    acc = b_ref[...] * jnp.ones((tx0.shape[0], 1), jnp.float32)
    for k, tx in enumerate((tx0, tx1, tx2, tx3)):
        acc = acc + jnp.dot(tx, w_ref[k], preferred_element_type=jnp.float32)
    hout = jnp.maximum(acc, 0.0)
    fo = hout.shape[1] // 2
    ha_ref[...] = hout[:, :fo]
    hb_ref[...] = hout[:, fo:]
    ga_ref[...] = -dv * hout[:, :fo]
    gb_ref[...] = -dv * hout[:, fo:]


def _m2_body(xa_ref, xb_ref, s1a_ref, s1b_ref, s2a_ref, s2b_ref,
             s3a_ref, s3b_ref, dv_ref, w_ref, b_ref, o_ref):
    n = pl.program_id(0)
    dv = dv_ref[...]
    fin = w_ref.shape[1]
    h = fin // 2
    acc = b_ref[...] * jnp.ones((xa_ref.shape[0], 1), jnp.float32)
    for c, (x_r, s1_r, s2_r, s3_r) in enumerate(
            ((xa_ref, s1a_ref, s2a_ref, s3a_ref),
             (xb_ref, s1b_ref, s2b_ref, s3b_ref))):
        tx0 = x_r[...]
        tx1 = dv * s1_r[...]
        tx2 = 2.0 * dv * s2_r[...] - tx0
        tx3 = 2.0 * dv * s3_r[...] - tx1
        w0 = w_ref[0, pl.ds(c * h, h), :]
        w1 = w_ref[1, pl.ds(c * h, h), :]
        w2 = w_ref[2, pl.ds(c * h, h), :]
        w3 = w_ref[3, pl.ds(c * h, h), :]
        acc = acc + jnp.dot(tx0, w0, preferred_element_type=jnp.float32)
        acc = acc + jnp.dot(tx1, w1, preferred_element_type=jnp.float32)
        acc = acc + jnp.dot(tx2, w2, preferred_element_type=jnp.float32)
        acc = acc + jnp.dot(tx3, w3, preferred_element_type=jnp.float32)
    hout = jnp.maximum(acc, 0.0)
    rows = n * NB + lax.broadcasted_iota(jnp.int32, (NB, 1), 0)
    hout = jnp.where(rows < N_NODES, hout, 0.0)
    colsum = jnp.sum(hout, axis=0, keepdims=True)

    @pl.when(n == 0)
    def _():
        o_ref[...] = colsum

    @pl.when(n > 0)
    def _():
        o_ref[...] = o_ref[...] + colsum


def _head_body(p_ref, w_ref, b_ref, o_ref):
    o_ref[...] = jnp.dot(p_ref[...], w_ref[...],
                         preferred_element_type=jnp.float32) + b_ref[...]


def _specA(fc=FC):
    return pl.BlockSpec((NB, fc), lambda n: (n, 0))


def _specB(fc=FC):
    return pl.BlockSpec((NB, fc), lambda n: (n + NBLK, 0))


def _dvspec():
    return pl.BlockSpec((NB, 1), lambda n: (n, 0))


def _f32(rows, cols=FC):
    return jax.ShapeDtypeStruct((rows, cols), jnp.float32)


def _premul0(x_pad, dinv_b):
    return pl.pallas_call(
        _premul0_body, grid=(NBLK,),
        in_specs=[_specA(), _dvspec()],
        out_specs=_specA(), out_shape=_f32(N_PAD),
    )(x_pad, dinv_b)


def _premul_sum(s_stk, dinv_b):
    return pl.pallas_call(
        _premul_sum_body, grid=(NBLK,),
        in_specs=[_specA(), _specB(), _dvspec()],
        out_specs=_specA(), out_shape=_f32(N_PAD),
    )(s_stk, s_stk, dinv_b)


def _q_sum(s_stk, x_pad, dinv_b):
    return pl.pallas_call(
        _q_sum_body, grid=(NBLK,),
        in_specs=[_specA(), _specB(), _specA(), _dvspec()],
        out_specs=_specA(), out_shape=_f32(N_PAD),
    )(s_stk, s_stk, x_pad, dinv_b)


def _premul_stk(s_stk, dinv_b):
    return pl.pallas_call(
        _premul_stk_body, grid=(2 * NBLK,),
        in_specs=[_specA(), pl.BlockSpec((NB, 1), lambda n: (n % NBLK, 0))],
        out_specs=_specA(), out_shape=_f32(2 * N_PAD),
    )(s_stk, dinv_b)


def _q_stk(s_stk, t_stk, dinv_b):
    return pl.pallas_call(
        _q_stk_body, grid=(2 * NBLK,),
        in_specs=[_specA(), _specA(),
                  pl.BlockSpec((NB, 1), lambda n: (n % NBLK, 0))],
        out_specs=_specA(), out_shape=_f32(2 * N_PAD),
    )(s_stk, t_stk, dinv_b)


def _m1(x_pad, s1, s2, s3, dinv_b, W, b):
    wspec = pl.BlockSpec(W.shape, lambda n: (0, 0, 0))
    bspec = pl.BlockSpec((1, 256), lambda n: (0, 0))
    return pl.pallas_call(
        _m1_body, grid=(NBLK,),
        in_specs=[_specA(), _specA(), _specB(), _specA(), _specB(),
                  _specA(), _specB(), _dvspec(), wspec, bspec],
        out_specs=[_specA()] * 4,
        out_shape=[_f32(N_PAD)] * 4,
    )(x_pad, s1, s1, s2, s2, s3, s3, dinv_b, W, b)


def _m2(h1a, h1b, t1, t2, t3, dinv_b, W, b):
    wspec = pl.BlockSpec(W.shape, lambda n: (0, 0, 0))
    bspec = pl.BlockSpec((1, 256), lambda n: (0, 0))
    return pl.pallas_call(
        _m2_body, grid=(NBLK,),
        in_specs=[_specA(), _specA(), _specA(), _specB(), _specA(), _specB(),
                  _specA(), _specB(), _dvspec(), wspec, bspec],
        out_specs=pl.BlockSpec((1, 256), lambda n: (0, 0)),
        out_shape=jax.ShapeDtypeStruct((1, 256), jnp.float32),
    )(h1a, h1b, t1, t1, t2, t2, t3, t3, dinv_b, W, b)


def kernel(x, edge_index, W1, b1, W2, b2, lin_W, lin_b):
    src = edge_index[0].astype(jnp.int32)
    dst = edge_index[1].astype(jnp.int32)
    srcp = jnp.pad(src, (0, E_PAD - E), constant_values=N_NODES)
    dstp = jnp.pad(dst, (0, E_PAD - E), constant_values=N_NODES)
    srcp2 = jnp.concatenate([srcp, srcp + N_PAD])
    x_pad = jnp.pad(x, ((0, N_PAD - N_NODES), (0, 0)))

    degs = _deg_kernel(srcp)
    dinv2 = pl.pallas_call(
        _dinv_body,
        out_shape=jax.ShapeDtypeStruct((HR, CHUNK), jnp.float32),
    )(degs[:N_PAD].reshape(HR, CHUNK), degs[N_PAD:].reshape(HR, CHUNK))
    dinv_b = dinv2.reshape(N_PAD, 1)

    g0 = _premul0(x_pad, dinv_b)
    s1 = _scatter_es(g0, srcp, dstp)
    g1 = _premul_sum(s1, dinv_b)
    s2 = _scatter_es(g1, srcp, dstp)
    g2 = _q_sum(s2, x_pad, dinv_b)
    s3 = _scatter_es(g2, srcp, dstp)
    h1a, h1b, g0a, g0b = _m1(x_pad, s1, s2, s3, dinv_b, W1,
                             b1.reshape(1, -1))

    h1s = jnp.concatenate([h1a, h1b], axis=0)
    g0s = jnp.concatenate([g0a, g0b], axis=0)
    t1 = _scatter_cs(g0s, srcp2, dstp)
    g1p = _premul_stk(t1, dinv_b)
    t2 = _scatter_cs(g1p, srcp2, dstp)
    g2p = _q_stk(t2, h1s, dinv_b)
    t3 = _scatter_cs(g2p, srcp2, dstp)
    pooled = _m2(h1a, h1b, t1, t2, t3, dinv_b, W2, b2.reshape(1, -1))

    out = pl.pallas_call(
        _head_body,
        out_shape=jax.ShapeDtypeStruct((1, lin_W.shape[1]), jnp.float32),
    )(pooled, lin_W, lin_b.reshape(1, -1))
    return (pooled, out)
